# baseline (speedup 1.0000x reference)
import functools

import jax
import jax.numpy as jnp
from jax import lax
from jax.experimental import pallas as pl
from jax.experimental.pallas import tpu as pltpu
from jax.experimental.pallas import tpu_sc as plsc

_NC = 2
_NS = 16
_NW = _NC * _NS
_L = 16

_CH = 128


def _index_body(x_ref, idx_ref, *, scale_max, stride):
    def one(v):
        mn = jnp.min(v)
        dn = jnp.max(v) - mn + 1e-8
        xn = jnp.clip((v - mn) / dn, 0.0, 1.0)
        return (xn * scale_max).astype(jnp.int32) * stride

    v = x_ref[...]
    ix = one(v[0])
    iy = one(v[1])
    idx_ref[...] = jnp.concatenate([ix[None], iy[None]], axis=0)


def _compute_indices(xt, scale_max, stride):
    return pl.pallas_call(
        functools.partial(_index_body, scale_max=scale_max, stride=stride),
        out_shape=jax.ShapeDtypeStruct(xt.shape, jnp.int32),
    )(xt)


def _make_sc_gather(max_len, d_half, n):
    rows_w = n // _NW
    n_pairs = rows_w // (2 * _CH)
    assert rows_w % (2 * _CH) == 0
    d_out = 2 * d_half
    mesh = plsc.VectorSubcoreMesh(core_axis_name="c", subcore_axis_name="s")

    @functools.partial(
        pl.kernel,
        mesh=mesh,
        out_type=jax.ShapeDtypeStruct((n, d_out), jnp.float32),
        compiler_params=pltpu.CompilerParams(needs_layout_passes=False),
        scratch_types=[
            pltpu.VMEM((max_len * d_half,), jnp.float32),
            pltpu.VMEM((rows_w,), jnp.int32),
            pltpu.VMEM((rows_w,), jnp.int32),
            pltpu.VMEM((_CH, d_out), jnp.float32),
            pltpu.VMEM((_CH, d_out), jnp.float32),
            pltpu.SemaphoreType.DMA,
            pltpu.SemaphoreType.DMA,
            pltpu.SemaphoreType.DMA,
            pltpu.SemaphoreType.DMA,
        ],
    )
    def sc_gather(
        pe_hbm,
        idx_hbm,
        out_hbm,
        pe_v,
        ixs_v,
        iys_v,
        out_v0,
        out_v1,
        sem_pe,
        sem_ix,
        sem_o0,
        sem_o1,
    ):
        wid = lax.axis_index("s") * _NC + lax.axis_index("c")
        row0 = wid * rows_w
        pe_cp = pltpu.async_copy(pe_hbm, pe_v, sem_pe)
        ix_cp = pltpu.async_copy(idx_hbm.at[pl.ds(row0, rows_w)], ixs_v, sem_ix)
        pltpu.async_copy(
            idx_hbm.at[pl.ds(n + row0, rows_w)], iys_v, sem_ix
        ).wait()
        ix_cp.wait()
        pe_cp.wait()

        def do_chunk(c, out_v, sem):
            @pl.when(c >= 2)
            def _():
                pltpu.make_async_copy(
                    out_v, out_hbm.at[pl.ds(row0 + c * _CH, _CH)], sem
                ).wait()

            @plsc.parallel_loop(0, _CH // _L, unroll=1)
            def group(g):
                ivx = ixs_v[pl.ds(c * _CH + g * _L, _L)]
                ivy = iys_v[pl.ds(c * _CH + g * _L, _L)]
                for k in range(_L):
                    ix = ivx[k]
                    iy = ivy[k]
                    row = g * _L + k
                    for cc in range(0, d_half, _L):
                        out_v[row, pl.ds(cc, _L)] = pe_v[pl.ds(ix + cc, _L)]
                    for cc in range(0, d_half, _L):
                        out_v[row, pl.ds(d_half + cc, _L)] = pe_v[
                            pl.ds(iy + cc, _L)
                        ]

            pltpu.async_copy(
                out_v, out_hbm.at[pl.ds(row0 + c * _CH, _CH)], sem
            )

        def pair(ci, carry):
            do_chunk(2 * ci, out_v0, sem_o0)
            do_chunk(2 * ci + 1, out_v1, sem_o1)
            return carry

        lax.fori_loop(0, n_pairs, pair, 0)

        last = 2 * n_pairs - 1
        pltpu.make_async_copy(
            out_v0, out_hbm.at[pl.ds(row0 + (last - 1) * _CH, _CH)], sem_o0
        ).wait()
        pltpu.make_async_copy(
            out_v1, out_hbm.at[pl.ds(row0 + last * _CH, _CH)], sem_o1
        ).wait()

    return sc_gather


def kernel(x, pe):
    n, two = x.shape
    max_len, d_half = pe.shape

    xt = x.T.reshape(two, n // 128, 128)
    idx3d = _compute_indices(xt, float(max_len - 1), d_half)
    idx_flat = idx3d.reshape(two * n)

    return _make_sc_gather(max_len, d_half, n)(pe.reshape(-1), idx_flat)


# speedup vs baseline: 1.0369x; 1.0369x over previous
import functools

import jax
import jax.numpy as jnp
from jax import lax
from jax.experimental import pallas as pl
from jax.experimental.pallas import tpu as pltpu
from jax.experimental.pallas import tpu_sc as plsc

_NC = 2
_NS = 16
_NW = _NC * _NS
_L = 16

_CH = 128


def _index_body(x_ref, idx_ref, *, scale_max, stride):
    def one(v):
        mn = jnp.min(v)
        dn = jnp.max(v) - mn + 1e-8
        xn = jnp.clip((v - mn) / dn, 0.0, 1.0)
        return (xn * scale_max).astype(jnp.int32) * stride

    v = x_ref[...]
    ix = one(v[0])
    iy = one(v[1])
    idx_ref[...] = jnp.concatenate([ix[None], iy[None]], axis=0)


def _compute_indices(xt, scale_max, stride):
    return pl.pallas_call(
        functools.partial(_index_body, scale_max=scale_max, stride=stride),
        out_shape=jax.ShapeDtypeStruct(xt.shape, jnp.int32),
    )(xt)


def _make_sc_gather(max_len, d_half, n):
    rows_w = n // _NW
    n_pairs = rows_w // (2 * _CH)
    assert rows_w % (2 * _CH) == 0
    d_out = 2 * d_half
    mesh = plsc.VectorSubcoreMesh(core_axis_name="c", subcore_axis_name="s")

    @functools.partial(
        pl.kernel,
        mesh=mesh,
        out_type=jax.ShapeDtypeStruct((n, d_out), jnp.float32),
        compiler_params=pltpu.CompilerParams(needs_layout_passes=False),
        scratch_types=[
            pltpu.VMEM((max_len * d_half,), jnp.float32),
            pltpu.VMEM((rows_w,), jnp.int32),
            pltpu.VMEM((rows_w,), jnp.int32),
            pltpu.VMEM((_CH, d_out), jnp.float32),
            pltpu.VMEM((_CH, d_out), jnp.float32),
            pltpu.SemaphoreType.DMA,
            pltpu.SemaphoreType.DMA,
            pltpu.SemaphoreType.DMA,
            pltpu.SemaphoreType.DMA,
        ],
    )
    def sc_gather(
        pe_hbm,
        idx_hbm,
        out_hbm,
        pe_v,
        ixs_v,
        iys_v,
        out_v0,
        out_v1,
        sem_pe,
        sem_ix,
        sem_o0,
        sem_o1,
    ):
        wid = lax.axis_index("s") * _NC + lax.axis_index("c")
        row0 = wid * rows_w
        pe_cp = pltpu.async_copy(pe_hbm, pe_v, sem_pe)
        ix_cp = pltpu.async_copy(idx_hbm.at[pl.ds(row0, rows_w)], ixs_v, sem_ix)
        pltpu.async_copy(
            idx_hbm.at[pl.ds(n + row0, rows_w)], iys_v, sem_ix
        ).wait()
        ix_cp.wait()
        pe_cp.wait()

        def do_chunk(c, out_v, sem):
            @pl.when(c >= 2)
            def _():
                pltpu.make_async_copy(
                    out_v, out_hbm.at[pl.ds(row0 + c * _CH, _CH)], sem
                ).wait()

            @plsc.parallel_loop(0, _CH // _L, unroll=2)
            def group(g):
                ivx = ixs_v[pl.ds(c * _CH + g * _L, _L)]
                ivy = iys_v[pl.ds(c * _CH + g * _L, _L)]
                for k in range(_L):
                    ix = ivx[k]
                    iy = ivy[k]
                    row = g * _L + k
                    for cc in range(0, d_half, _L):
                        out_v[row, pl.ds(cc, _L)] = pe_v[pl.ds(ix + cc, _L)]
                    for cc in range(0, d_half, _L):
                        out_v[row, pl.ds(d_half + cc, _L)] = pe_v[
                            pl.ds(iy + cc, _L)
                        ]

            pltpu.async_copy(
                out_v, out_hbm.at[pl.ds(row0 + c * _CH, _CH)], sem
            )

        def pair(ci, carry):
            do_chunk(2 * ci, out_v0, sem_o0)
            do_chunk(2 * ci + 1, out_v1, sem_o1)
            return carry

        lax.fori_loop(0, n_pairs, pair, 0)

        last = 2 * n_pairs - 1
        pltpu.make_async_copy(
            out_v0, out_hbm.at[pl.ds(row0 + (last - 1) * _CH, _CH)], sem_o0
        ).wait()
        pltpu.make_async_copy(
            out_v1, out_hbm.at[pl.ds(row0 + last * _CH, _CH)], sem_o1
        ).wait()

    return sc_gather


def kernel(x, pe):
    n, two = x.shape
    max_len, d_half = pe.shape

    xt = x.T.reshape(two, n // 128, 128)
    idx3d = _compute_indices(xt, float(max_len - 1), d_half)
    idx_flat = idx3d.reshape(two * n)

    return _make_sc_gather(max_len, d_half, n)(pe.reshape(-1), idx_flat)
